# baseline (speedup 1.0000x reference)
import functools

import jax
import jax.numpy as jnp
from jax import lax
from jax.experimental import pallas as pl
from jax.experimental.pallas import tpu as pltpu
from jax.experimental.pallas import tpu_sc as plsc

_B, _T, _V = 64, 32, 32768
_N = _B * _T
_LN2 = 0.6931471805599453


def _sc_loss_partials(rows_view, target, maskf):
    info = plsc.get_sparse_core_info()
    nc, ns = info.num_cores, info.num_subcores
    nw = nc * ns
    per_w = _N // nw
    ngrp = per_w // 16
    mesh = plsc.VectorSubcoreMesh(core_axis_name="c", subcore_axis_name="s")

    @functools.partial(
        pl.kernel,
        mesh=mesh,
        out_type=jax.ShapeDtypeStruct((nw * 32,), jnp.float32),
        compiler_params=pltpu.CompilerParams(needs_layout_passes=False),
        scratch_types=[
            pltpu.VMEM((8, _T), jnp.int32),
            pltpu.VMEM((8, _T), jnp.float32),
            pltpu.VMEM((per_w,), jnp.int32),
            pltpu.VMEM((per_w, 128), jnp.float32),
            pltpu.VMEM((16,), jnp.float32),
            pltpu.VMEM((16,), jnp.float32),
            pltpu.VMEM((16, 16), jnp.float32),
            pltpu.VMEM_SHARED((16, 16), jnp.float32),
            pltpu.VMEM_SHARED((16, 16), jnp.float32),
            pltpu.SemaphoreType.DMA,
        ],
    )
    def body(
        rows_hbm, tgt_hbm, msk_hbm, out_hbm,
        tgt_v, msk_v, idx_v, buf_v, accbuf, cntbuf, gath_v,
        shared_s, shared_c, sem,
    ):
        cid = lax.axis_index("c")
        sid = lax.axis_index("s")
        wid = sid * nc + cid
        base = wid * per_w
        b0 = pl.multiple_of(wid // 4 * 8, 8)
        pltpu.sync_copy(tgt_hbm.at[pl.ds(b0, 8), :], tgt_v)
        pltpu.sync_copy(msk_hbm.at[pl.ds(b0, 8), :], msk_v)
        lane = lax.iota(jnp.int32, 16)
        for j in range(ngrp):
            row_local = 2 * (wid % 4) + j // 2
            tv = tgt_v[row_local, pl.ds(j % 2 * 16, 16)]
            e16 = base + j * 16 + lane
            b16 = jnp.right_shift(e16, 5)
            tpos = jnp.bitwise_and(e16, 31)
            const = (
                b16 * 8192
                + jnp.right_shift(tpos, 3) * 2048
                + jnp.bitwise_and(tpos, 7)
            )
            idx_v[pl.ds(j * 16, 16)] = const + jnp.right_shift(tv, 7) * 8
        pltpu.async_copy(rows_hbm.at[idx_v], buf_v, sem).wait()
        acc_s = jnp.zeros((16,), jnp.float32)
        acc_c = jnp.zeros((16,), jnp.float32)
        for j in range(ngrp):
            row_local = 2 * (wid % 4) + j // 2
            tv = tgt_v[row_local, pl.ds(j % 2 * 16, 16)]
            mv = msk_v[row_local, pl.ds(j % 2 * 16, 16)]
            col = jnp.bitwise_and(tv, 127)
            p16 = plsc.load_gather(buf_v, [j * 16 + lane, col])
            bits = plsc.bitcast(p16, jnp.int32)
            e = (jnp.right_shift(bits, 23) - 127).astype(jnp.float32)
            mant = plsc.bitcast(
                jnp.bitwise_or(
                    jnp.bitwise_and(bits, 0x007FFFFF), 0x3F800000
                ),
                jnp.float32,
            )
            s = (mant - 1.0) / (mant + 1.0)
            s2 = s * s
            ln_m = s * (
                2.0 + s2 * (2.0 / 3.0 + s2 * (2.0 / 5.0 + s2 * (2.0 / 7.0)))
            )
            lnp = e * _LN2 + ln_m
            acc_s = acc_s - lnp * mv
            acc_c = acc_c + mv
        accbuf[...] = acc_s
        cntbuf[...] = acc_c
        pltpu.sync_copy(accbuf, out_hbm.at[pl.ds(wid * 32, 16)])
        pltpu.sync_copy(cntbuf, out_hbm.at[pl.ds(wid * 32 + 16, 16)])

    return body(rows_view, target, maskf)


def kernel(input, target, mask):
    rows_view = (
        input.reshape(_B, _T // 8, 8, _V // 128, 128)
        .transpose(0, 1, 3, 2, 4)
        .reshape(_B * (_T // 8) * (_V // 128) * 8, 128)
    )
    partials = _sc_loss_partials(rows_view, target, mask.astype(jnp.float32))
    pr = partials.reshape(32, 2, 16)
    return pr[:, 0].sum() / pr[:, 1].sum()


# speedup vs baseline: 1.1388x; 1.1388x over previous
import functools

import jax
import jax.numpy as jnp
from jax import lax
from jax.experimental import pallas as pl
from jax.experimental.pallas import tpu as pltpu
from jax.experimental.pallas import tpu_sc as plsc

_B, _T, _V = 64, 32, 32768
_N = _B * _T


def _sc_gather(rows_view, tgt_flat):
    info = plsc.get_sparse_core_info()
    nc, ns = info.num_cores, info.num_subcores
    nw = nc * ns
    per_w = _N // nw
    ngrp = per_w // 16
    mesh = plsc.VectorSubcoreMesh(core_axis_name="c", subcore_axis_name="s")

    @functools.partial(
        pl.kernel,
        mesh=mesh,
        out_type=jax.ShapeDtypeStruct((_N,), jnp.float32),
        compiler_params=pltpu.CompilerParams(needs_layout_passes=False),
        scratch_types=[
            pltpu.VMEM((per_w,), jnp.int32),
            pltpu.VMEM((per_w,), jnp.int32),
            pltpu.VMEM((per_w, 128), jnp.float32),
            pltpu.VMEM((per_w,), jnp.float32),
            pltpu.SemaphoreType.DMA,
        ],
    )
    def body(rows_hbm, tgt_hbm, out_hbm, tgt_v, idx_v, buf_v, vals_v, sem):
        wid = lax.axis_index("s") * nc + lax.axis_index("c")
        base = wid * per_w
        pltpu.sync_copy(tgt_hbm.at[pl.ds(base, per_w)], tgt_v)
        lane = lax.iota(jnp.int32, 16)
        for j in range(ngrp):
            tv = tgt_v[pl.ds(j * 16, 16)]
            e16 = base + j * 16 + lane
            b16 = jnp.right_shift(e16, 5)
            tpos = jnp.bitwise_and(e16, 31)
            const = (
                b16 * 8192
                + jnp.right_shift(tpos, 3) * 2048
                + jnp.bitwise_and(tpos, 7)
            )
            idx_v[pl.ds(j * 16, 16)] = const + jnp.right_shift(tv, 7) * 8
        pltpu.async_copy(rows_hbm.at[idx_v], buf_v, sem).wait()
        for j in range(ngrp):
            tv = tgt_v[pl.ds(j * 16, 16)]
            col = jnp.bitwise_and(tv, 127)
            vals_v[pl.ds(j * 16, 16)] = plsc.load_gather(
                buf_v, [j * 16 + lane, col]
            )
        pltpu.sync_copy(vals_v, out_hbm.at[pl.ds(base, per_w)])

    return body(rows_view, tgt_flat)


def _tc_loss(vals_ref, mask_ref, out_ref):
    v = vals_ref[...]
    m = mask_ref[...]
    ce = -jnp.log(v)
    out_ref[0, 0] = jnp.sum(ce * m) / jnp.sum(m)


def kernel(input, target, mask):
    rows_view = (
        input.reshape(_B, _T // 8, 8, _V // 128, 128)
        .transpose(0, 1, 3, 2, 4)
        .reshape(_B * (_T // 8) * (_V // 128) * 8, 128)
    )
    tgt = target.reshape(-1)
    vals = _sc_gather(rows_view, tgt)
    mask_f = mask.reshape(16, 128).astype(jnp.float32)
    out = pl.pallas_call(
        _tc_loss,
        out_shape=jax.ShapeDtypeStruct((1, 1), jnp.float32),
        out_specs=pl.BlockSpec(memory_space=pltpu.SMEM),
    )(vals.reshape(16, 128), mask_f)
    return out[0, 0]
